# baseline (speedup 1.0000x reference)
import functools

import jax
import jax.numpy as jnp
from jax import lax
from jax.experimental import pallas as pl
from jax.experimental.pallas import tpu as pltpu
from jax.experimental.pallas import tpu_sc as plsc

F32 = jnp.float32
NSUB = 16
KCH = 128
HALF = 128
ROWB = 400


def _full(shape):
    return pl.BlockSpec(shape, lambda i: (0, 0))


def _rows(width):
    return pl.BlockSpec((ROWB, width), lambda i: (i, 0))


def _t1_body(xb, xs, Web, beb, Wes, bes, Wsb, Wbs, Wss, Wrb, Wrs,
             ysb_lo, ybs_lo, yss_lo, rb, rs):
    h_b = jnp.maximum(
        jnp.dot(xb[...], Web[...], preferred_element_type=F32) + beb[...], 0.0)
    h_s = jnp.maximum(
        jnp.dot(xs[...], Wes[...], preferred_element_type=F32) + bes[...], 0.0)
    ysb_lo[...] = jnp.dot(h_s, Wsb[...], preferred_element_type=F32)
    ybs_lo[...] = jnp.dot(h_b, Wbs[...], preferred_element_type=F32)
    yss_lo[...] = jnp.dot(h_s, Wss[...], preferred_element_type=F32)
    rb[...] = jnp.dot(h_b, Wrb[...], preferred_element_type=F32)
    rs[...] = jnp.dot(h_s, Wrs[...], preferred_element_type=F32)


def _t1(xb, xs, Web, beb, Wes, bes, Wsb, Wbs, Wss, Wrb, Wrs):
    n, db = xb.shape
    ds_ = xs.shape[1]
    h = Web.shape[1]
    grid = n // ROWB
    return pl.pallas_call(
        _t1_body,
        grid=(grid,),
        in_specs=[
            _rows(db), _rows(ds_),
            _full((db, h)), _full((1, h)), _full((ds_, h)), _full((1, h)),
            _full((h, h)), _full((h, h)), _full((h, h)),
            _full((h, h)), _full((h, h)),
        ],
        out_specs=[_rows(h)] * 5,
        out_shape=[jax.ShapeDtypeStruct((n, h), F32)] * 5,
    )(xb, xs, Web, beb, Wes, bes, Wsb, Wbs, Wss, Wrb, Wrs)


def _pad_rows(n):
    return ((n + NSUB * 8 - 1) // (NSUB * 8)) * (NSUB * 8)


def _make_agg(n_dst, n_edges):
    e_per_sub = n_edges // NSUB
    nch = e_per_sub // KCH
    assert e_per_sub * NSUB == n_edges and nch * KCH == e_per_sub
    assert nch % 4 == 0
    npad = _pad_rows(n_dst)
    assert npad > n_dst
    rps = npad // NSUB
    mesh = plsc.VectorSubcoreMesh(core_axis_name="c", subcore_axis_name="s")

    def body(y2, src, dst, zrows, out, *refs):
        idx = refs[0:4]
        didx = refs[4:8]
        rows = refs[8:10]
        accum = refs[10]
        si = refs[11:15]
        sg = refs[15:17]
        ss = refs[17:19]
        c = lax.axis_index("c")
        s = lax.axis_index("s")
        r0 = s * rps
        e0 = s * e_per_sub
        sbase = c * n_edges + e0
        pltpu.sync_copy(zrows.at[pl.ds(r0, rps)], accum.at[pl.ds(r0, rps)])
        plsc.subcore_barrier()

        def load_idx(i, m):
            pltpu.async_copy(src.at[pl.ds(sbase + i * KCH, KCH)], idx[m],
                             si[m])
            pltpu.async_copy(dst.at[pl.ds(e0 + i * KCH, KCH)], didx[m],
                             si[m])

        def wait_idx(m):
            pltpu.make_async_copy(src.at[pl.ds(0, KCH)], idx[m],
                                  si[m]).wait()
            pltpu.make_async_copy(dst.at[pl.ds(0, KCH)], didx[m],
                                  si[m]).wait()

        def gather(m):
            pltpu.async_copy(y2.at[idx[m]], rows[m % 2], sg[m % 2])

        def wait_gather(m):
            pltpu.make_async_copy(y2.at[idx[m]], rows[m % 2],
                                  sg[m % 2]).wait()

        def drain_scatter(m):
            pltpu.make_async_copy(rows[m % 2], accum.at[didx[m]],
                                  ss[m % 2]).wait()

        load_idx(0, 0)
        load_idx(1, 1)
        wait_idx(0)
        gather(0)

        def phase(j, m):
            m1 = (m + 1) % 4
            @pl.when(j + 1 < nch)
            def _():
                wait_idx(m1)
            @pl.when(j >= 1)
            def _():
                drain_scatter((m + 3) % 4)
            @pl.when(j + 1 < nch)
            def _():
                gather(m1)
            wait_gather(m)
            pltpu.async_copy(rows[m % 2], accum.at[didx[m]], ss[m % 2],
                             add=True)
            @pl.when(j + 2 < nch)
            def _():
                load_idx(j + 2, (m + 2) % 4)

        def group(g, carry):
            for m in range(4):
                phase(4 * g + m, m)
            return carry

        lax.fori_loop(0, nch // 4, group, 0)
        drain_scatter((nch - 1) % 4)
        plsc.subcore_barrier()
        o0 = c * npad + r0
        pltpu.sync_copy(accum.at[pl.ds(r0, rps)], out.at[pl.ds(o0, rps)])

    return pl.kernel(
        body, mesh=mesh,
        out_type=[jax.ShapeDtypeStruct((2 * npad, HALF), F32)],
        scratch_types=(
            [pltpu.VMEM((KCH,), jnp.int32) for _ in range(8)]
            + [pltpu.VMEM((KCH, HALF), F32) for _ in range(2)]
            + [pltpu.VMEM_SHARED((npad, HALF), F32)]
            + [pltpu.SemaphoreType.DMA] * 4
            + [pltpu.SemaphoreType.DMA] * 2
            + [pltpu.SemaphoreType.DMA] * 2
        ))


def _make_l0(npadm, n_edges):
    e_per_sub = n_edges // NSUB
    nch = e_per_sub // KCH
    assert nch * KCH * NSUB == n_edges and nch % 4 == 0
    e_half = n_edges // 2
    ec_per_sub = e_half // NSUB
    ncc = ec_per_sub // KCH
    assert ncc * KCH * NSUB * 2 == n_edges and ncc % 4 == 0
    rps = npadm // NSUB
    mesh = plsc.VectorSubcoreMesh(core_axis_name="c", subcore_axis_name="s")

    def body(y0, y1t, y2t, sx0, sx1, sx2, d0, d1, d2, zrows, ones_h,
             o0, o1, o2, co0, co1, co2, *refs):
        idx = refs[0:4]
        didx = refs[4:8]
        rows = refs[8:10]
        accum = refs[10]
        ones_v = refs[11]
        si = refs[12:16]
        sg = refs[16:18]
        ss = refs[18:20]
        c = lax.axis_index("c")
        s = lax.axis_index("s")
        r0 = s * rps
        pltpu.sync_copy(ones_h, ones_v)

        def zero():
            pltpu.sync_copy(zrows.at[pl.ds(r0, rps)],
                            accum.at[pl.ds(r0, rps)])
            plsc.subcore_barrier()

        def flush(out):
            plsc.subcore_barrier()
            pltpu.sync_copy(accum.at[pl.ds(r0, rps)],
                            out.at[pl.ds(c * npadm + r0, rps)])

        def agg_rel(y2, src, dst, out):
            zero()
            e0 = s * e_per_sub
            sbase = c * n_edges + e0

            def load_idx(i, m):
                pltpu.async_copy(src.at[pl.ds(sbase + i * KCH, KCH)],
                                 idx[m], si[m])
                pltpu.async_copy(dst.at[pl.ds(e0 + i * KCH, KCH)],
                                 didx[m], si[m])

            def wait_idx(m):
                pltpu.make_async_copy(src.at[pl.ds(0, KCH)], idx[m],
                                      si[m]).wait()
                pltpu.make_async_copy(dst.at[pl.ds(0, KCH)], didx[m],
                                      si[m]).wait()

            def gather(m):
                pltpu.async_copy(y2.at[idx[m]], rows[m % 2], sg[m % 2])

            def wait_gather(m):
                pltpu.make_async_copy(y2.at[idx[m]], rows[m % 2],
                                      sg[m % 2]).wait()

            def drain_scatter(m):
                pltpu.make_async_copy(rows[m % 2], accum.at[didx[m]],
                                      ss[m % 2]).wait()

            load_idx(0, 0)
            load_idx(1, 1)
            wait_idx(0)
            gather(0)

            def phase(j, m):
                m1 = (m + 1) % 4
                @pl.when(j + 1 < nch)
                def _():
                    wait_idx(m1)
                @pl.when(j >= 1)
                def _():
                    drain_scatter((m + 3) % 4)
                @pl.when(j + 1 < nch)
                def _():
                    gather(m1)
                wait_gather(m)
                pltpu.async_copy(rows[m % 2], accum.at[didx[m]],
                                 ss[m % 2], add=True)
                @pl.when(j + 2 < nch)
                def _():
                    load_idx(j + 2, (m + 2) % 4)

            def group(g, carry):
                for m in range(4):
                    phase(4 * g + m, m)
                return carry

            lax.fori_loop(0, nch // 4, group, 0)
            drain_scatter((nch - 1) % 4)
            flush(out)

        def cnt_rel(dst, out):
            zero()
            e0 = c * e_half + s * ec_per_sub

            def load_didx(i, m):
                pltpu.async_copy(dst.at[pl.ds(e0 + i * KCH, KCH)],
                                 didx[m], si[m])

            load_didx(0, 0)
            load_didx(1, 1)

            def phase(j, m):
                mp = (m + 2) % 4
                @pl.when(j >= 2)
                def _():
                    pltpu.make_async_copy(ones_v, accum.at[didx[mp]],
                                          ss[mp % 2]).wait()
                @pl.when(j + 2 < ncc)
                def _():
                    load_didx(j + 2, mp)
                pltpu.make_async_copy(dst.at[pl.ds(0, KCH)], didx[m],
                                      si[m]).wait()
                pltpu.async_copy(ones_v, accum.at[didx[m]], ss[m % 2],
                                 add=True)

            def group(g, carry):
                for m in range(4):
                    phase(4 * g + m, m)
                return carry

            lax.fori_loop(0, ncc // 4, group, 0)
            for m in ((ncc - 2) % 4, (ncc - 1) % 4):
                pltpu.make_async_copy(ones_v, accum.at[didx[m]],
                                      ss[m % 2]).wait()
            flush(out)

        agg_rel(y0, sx0, d0, o0)
        agg_rel(y1t, sx1, d1, o1)
        agg_rel(y2t, sx2, d2, o2)
        cnt_rel(d0, co0)
        cnt_rel(d1, co1)
        cnt_rel(d2, co2)

    return pl.kernel(
        body, mesh=mesh,
        out_type=[jax.ShapeDtypeStruct((2 * npadm, HALF), F32)] * 6,
        scratch_types=(
            [pltpu.VMEM((KCH,), jnp.int32) for _ in range(8)]
            + [pltpu.VMEM((KCH, HALF), F32) for _ in range(2)]
            + [pltpu.VMEM_SHARED((npadm, HALF), F32)]
            + [pltpu.VMEM((KCH, HALF), F32)]
            + [pltpu.SemaphoreType.DMA] * 4
            + [pltpu.SemaphoreType.DMA] * 2
            + [pltpu.SemaphoreType.DMA] * 2
        ))


def _t2_body(asb_lo, asb_hi, abs_lo, abs_hi, ass_lo, ass_hi,
             csb, cbs, css, rb, rs, b_sb, b_bs, b_ss, Wl1, Wr1,
             y1o, r1b):
    dsb = jnp.maximum(csb[...][:, 0:1], 1.0)
    dbs = jnp.maximum(cbs[...][:, 0:1], 1.0)
    dss = jnp.maximum(css[...][:, 0:1], 1.0)
    m_sb = jnp.concatenate([asb_lo[...], asb_hi[...]], axis=1) / dsb
    m_bs = jnp.concatenate([abs_lo[...], abs_hi[...]], axis=1) / dbs
    m_ss = jnp.concatenate([ass_lo[...], ass_hi[...]], axis=1) / dss
    h_b1 = jnp.maximum(m_sb + b_sb[...] + rb[...], 0.0)
    h_s1 = jnp.maximum(m_bs + m_ss + b_bs[...] + b_ss[...] + rs[...], 0.0)
    y1o[...] = jnp.dot(h_s1, Wl1[...], preferred_element_type=F32)
    r1b[...] = jnp.dot(h_b1, Wr1[...], preferred_element_type=F32)


def _t2(asb_lo, asb_hi, abs_lo, abs_hi, ass_lo, ass_hi, csb, cbs, css,
        rb, rs, b_sb, b_bs, b_ss, Wl1, Wr1):
    n = rb.shape[0]
    h = rb.shape[1]
    grid = n // ROWB
    half = pl.BlockSpec((ROWB, HALF), lambda i: (i, 0))
    cnt = pl.BlockSpec((ROWB, 16), lambda i: (i, 0))
    return pl.pallas_call(
        _t2_body,
        grid=(grid,),
        in_specs=[half] * 6 + [cnt] * 3 + [_rows(h), _rows(h)]
        + [_full((1, h))] * 3 + [_full((h, h))] * 2,
        out_specs=[_rows(h), _rows(h)],
        out_shape=[jax.ShapeDtypeStruct((n, h), F32)] * 2,
    )(asb_lo, asb_hi, abs_lo, abs_hi, ass_lo, ass_hi, csb, cbs, css,
      rb, rs, b_sb, b_bs, b_ss, Wl1, Wr1)


def _t3_body(a1lo, a1hi, csb, r1b, b_sb1, W1, b1, w2row, b2, out):
    d = jnp.maximum(csb[...][:, 0:1], 1.0)
    nb1 = jnp.concatenate([a1lo[...], a1hi[...]], axis=1) / d \
        + b_sb1[...] + r1b[...]
    hidden = jnp.maximum(
        jnp.dot(nb1, W1[...], preferred_element_type=F32) + b1[...], 0.0)
    out[...] = jnp.sum(hidden * w2row[...], axis=1, keepdims=True) + b2[...]


def _t3(a1lo, a1hi, csb, r1b, b_sb1, W1, b1, w2row, b2):
    n = r1b.shape[0]
    h = r1b.shape[1]
    hh = W1.shape[1]
    grid = n // ROWB
    half = pl.BlockSpec((ROWB, HALF), lambda i: (i, 0))
    cnt = pl.BlockSpec((ROWB, 16), lambda i: (i, 0))
    return pl.pallas_call(
        _t3_body,
        grid=(grid,),
        in_specs=[half, half, cnt, _rows(h), _full((1, h)),
                  _full((h, hh)), _full((1, hh)), _full((1, hh)),
                  _full((1, 1))],
        out_specs=pl.BlockSpec((ROWB, 1), lambda i: (i, 0)),
        out_shape=jax.ShapeDtypeStruct((n, 1), F32),
    )(a1lo, a1hi, csb, r1b, b_sb1, W1, b1, w2row, b2)


def kernel(x_bridge, x_street, edge_index_street_to_bridge,
           edge_index_bridge_to_street, edge_index_street_to_street,
           We_b, be_b, We_s, be_s,
           Wl0_sb, bl0_sb, Wr0_sb, Wl0_bs, bl0_bs, Wr0_bs,
           Wl0_ss, bl0_ss, Wr0_ss,
           Wl1_sb, bl1_sb, Wr1_sb, Wl1_bs, bl1_bs, Wr1_bs,
           Wl1_ss, bl1_ss, Wr1_ss,
           W1, b1, W2, b2):
    n_b = x_bridge.shape[0]
    n_s = x_street.shape[0]
    n_e = edge_index_street_to_bridge.shape[1]
    h = We_b.shape[1]

    ep = -(-n_e // (2 * NSUB * KCH)) * (2 * NSUB * KCH)
    pad = ep - n_e
    ei32 = lambda e: e.astype(jnp.int32)
    def sx(e):
        s = jnp.concatenate([ei32(e[0]), jnp.zeros((pad,), jnp.int32)])
        return jnp.concatenate([s * 2, s * 2 + 1])

    def dx(e, trash):
        return jnp.concatenate(
            [ei32(e[1]), jnp.full((pad,), trash, jnp.int32)])

    tr_b = n_b
    tr_s = n_s
    src_sb = sx(edge_index_street_to_bridge)
    dst_sb = dx(edge_index_street_to_bridge, tr_b)
    src_bs = sx(edge_index_bridge_to_street)
    dst_bs = dx(edge_index_bridge_to_street, tr_s)
    src_ss = sx(edge_index_street_to_street)
    dst_ss = dx(edge_index_street_to_street, tr_s)

    row2 = lambda v: v.reshape(1, -1)
    Wr0_s = Wr0_bs + Wr0_ss

    ysb, ybs, yss, rb, rs = _t1(
        x_bridge, x_street, We_b, row2(be_b), We_s, row2(be_s),
        Wl0_sb, Wl0_bs, Wl0_ss, Wr0_sb, Wr0_s)

    npad_b = _pad_rows(n_b)
    npad_s = _pad_rows(n_s)
    zrows = jnp.zeros((max(npad_b, npad_s), HALF), F32)
    ones_h = jnp.ones((KCH, HALF), F32)
    tab = lambda y: y.reshape(-1, HALF)

    npadm = max(npad_b, npad_s)
    l0 = _make_l0(npadm, ep)
    agg_b = _make_agg(n_b, ep)

    asb, abs_, ass, csb_p, cbs_p, css_p = l0(
        tab(ysb), tab(ybs), tab(yss), src_sb, src_bs, src_ss,
        dst_sb, dst_bs, dst_ss, zrows, ones_h)

    comb = lambda p, n: (p[:n, :16] + p[npadm:npadm + n, :16])
    csb = comb(csb_p, n_b)
    cbs = comb(cbs_p, n_s)
    css = comb(css_p, n_s)

    lo = lambda a, npad, n: a[:n]
    hi = lambda a, npad, n: a[npad:npad + n]

    y1, r1b = _t2(lo(asb, npadm, n_b), hi(asb, npadm, n_b),
                  lo(abs_, npadm, n_s), hi(abs_, npadm, n_s),
                  lo(ass, npadm, n_s), hi(ass, npadm, n_s),
                  csb, cbs, css, rb, rs,
                  row2(bl0_sb), row2(bl0_bs), row2(bl0_ss),
                  Wl1_sb, Wr1_sb)

    a1 = agg_b(tab(y1), src_sb, dst_sb, zrows[:npad_b])[0]

    return _t3(lo(a1, npad_b, n_b), hi(a1, npad_b, n_b), csb, r1b,
               row2(bl1_sb), W1, row2(b1), W2.reshape(1, -1),
               b2.reshape(1, 1))


# speedup vs baseline: 1.0294x; 1.0294x over previous
import functools

import jax
import jax.numpy as jnp
from jax import lax
from jax.experimental import pallas as pl
from jax.experimental.pallas import tpu as pltpu
from jax.experimental.pallas import tpu_sc as plsc

F32 = jnp.float32
NSUB = 16
KCH = 128
HALF = 128
ROWB = 400


def _full(shape):
    return pl.BlockSpec(shape, lambda i: (0, 0))


def _rows(width):
    return pl.BlockSpec((ROWB, width), lambda i: (i, 0))


def _t1_body(xb, xs, Web, beb, Wes, bes, Wsb, Wbs, Wss, Wrb, Wrs,
             ysb_lo, ybs_lo, yss_lo, rb, rs):
    h_b = jnp.maximum(
        jnp.dot(xb[...], Web[...], preferred_element_type=F32) + beb[...], 0.0)
    h_s = jnp.maximum(
        jnp.dot(xs[...], Wes[...], preferred_element_type=F32) + bes[...], 0.0)
    ysb_lo[...] = jnp.dot(h_s, Wsb[...], preferred_element_type=F32)
    ybs_lo[...] = jnp.dot(h_b, Wbs[...], preferred_element_type=F32)
    yss_lo[...] = jnp.dot(h_s, Wss[...], preferred_element_type=F32)
    rb[...] = jnp.dot(h_b, Wrb[...], preferred_element_type=F32)
    rs[...] = jnp.dot(h_s, Wrs[...], preferred_element_type=F32)


def _t1(xb, xs, Web, beb, Wes, bes, Wsb, Wbs, Wss, Wrb, Wrs):
    n, db = xb.shape
    ds_ = xs.shape[1]
    h = Web.shape[1]
    grid = n // ROWB
    return pl.pallas_call(
        _t1_body,
        grid=(grid,),
        in_specs=[
            _rows(db), _rows(ds_),
            _full((db, h)), _full((1, h)), _full((ds_, h)), _full((1, h)),
            _full((h, h)), _full((h, h)), _full((h, h)),
            _full((h, h)), _full((h, h)),
        ],
        out_specs=[_rows(h)] * 5,
        out_shape=[jax.ShapeDtypeStruct((n, h), F32)] * 5,
    )(xb, xs, Web, beb, Wes, bes, Wsb, Wbs, Wss, Wrb, Wrs)


def _pad_rows(n):
    return ((n + NSUB * 8 - 1) // (NSUB * 8)) * (NSUB * 8)


def _make_agg(n_dst, n_edges):
    e_per_sub = n_edges // NSUB
    nch = e_per_sub // KCH
    assert e_per_sub * NSUB == n_edges and nch * KCH == e_per_sub
    assert nch % 4 == 0
    npad = _pad_rows(n_dst)
    assert npad > n_dst
    rps = npad // NSUB
    mesh = plsc.VectorSubcoreMesh(core_axis_name="c", subcore_axis_name="s")

    def body(y2, src, dst, zrows, out, *refs):
        idx = refs[0:4]
        didx = refs[4:8]
        rows = refs[8:10]
        accum = refs[10]
        si = refs[11:15]
        sg = refs[15:17]
        ss = refs[17:19]
        c = lax.axis_index("c")
        s = lax.axis_index("s")
        r0 = s * rps
        e0 = s * e_per_sub
        sbase = c * n_edges + e0
        pltpu.sync_copy(zrows.at[pl.ds(r0, rps)], accum.at[pl.ds(r0, rps)])
        plsc.subcore_barrier()

        def load_idx(i, m):
            pltpu.async_copy(src.at[pl.ds(sbase + i * KCH, KCH)], idx[m],
                             si[m])
            pltpu.async_copy(dst.at[pl.ds(e0 + i * KCH, KCH)], didx[m],
                             si[m])

        def wait_idx(m):
            pltpu.make_async_copy(src.at[pl.ds(0, KCH)], idx[m],
                                  si[m]).wait()
            pltpu.make_async_copy(dst.at[pl.ds(0, KCH)], didx[m],
                                  si[m]).wait()

        def gather(m):
            pltpu.async_copy(y2.at[idx[m]], rows[m % 2], sg[m % 2])

        def wait_gather(m):
            pltpu.make_async_copy(y2.at[idx[m]], rows[m % 2],
                                  sg[m % 2]).wait()

        def drain_scatter(m):
            pltpu.make_async_copy(rows[m % 2], accum.at[didx[m]],
                                  ss[m % 2]).wait()

        load_idx(0, 0)
        load_idx(1, 1)
        wait_idx(0)
        gather(0)

        def phase(j, m):
            m1 = (m + 1) % 4
            @pl.when(j + 1 < nch)
            def _():
                wait_idx(m1)
            @pl.when(j >= 1)
            def _():
                drain_scatter((m + 3) % 4)
            @pl.when(j + 1 < nch)
            def _():
                gather(m1)
            wait_gather(m)
            pltpu.async_copy(rows[m % 2], accum.at[didx[m]], ss[m % 2],
                             add=True)
            @pl.when(j + 2 < nch)
            def _():
                load_idx(j + 2, (m + 2) % 4)

        def group(g, carry):
            for m in range(4):
                phase(4 * g + m, m)
            return carry

        lax.fori_loop(0, nch // 4, group, 0)
        drain_scatter((nch - 1) % 4)
        plsc.subcore_barrier()
        o0 = c * npad + r0
        pltpu.sync_copy(accum.at[pl.ds(r0, rps)], out.at[pl.ds(o0, rps)])

    return pl.kernel(
        body, mesh=mesh,
        out_type=[jax.ShapeDtypeStruct((2 * npad, HALF), F32)],
        scratch_types=(
            [pltpu.VMEM((KCH,), jnp.int32) for _ in range(8)]
            + [pltpu.VMEM((KCH, HALF), F32) for _ in range(2)]
            + [pltpu.VMEM_SHARED((npad, HALF), F32)]
            + [pltpu.SemaphoreType.DMA] * 4
            + [pltpu.SemaphoreType.DMA] * 2
            + [pltpu.SemaphoreType.DMA] * 2
        ))


def _make_cnt(n_dst, n_edges):
    e_half = n_edges // 2
    e_per_sub = e_half // NSUB
    nch = e_per_sub // KCH
    assert e_per_sub * NSUB * 2 == n_edges and nch * KCH == e_per_sub
    assert nch % 4 == 0
    npad = _pad_rows(n_dst)
    assert npad > n_dst
    rps = npad // NSUB
    mesh = plsc.VectorSubcoreMesh(core_axis_name="c", subcore_axis_name="s")

    def body(dst, zrows, ones_h, out, *refs):
        didx = refs[0:4]
        ones_v = refs[4]
        cacc = refs[5]
        si = refs[6:10]
        ss = refs[10:14]
        c = lax.axis_index("c")
        s = lax.axis_index("s")
        r0 = s * rps
        e0 = c * e_half + s * e_per_sub
        pltpu.sync_copy(zrows.at[pl.ds(r0, rps)], cacc.at[pl.ds(r0, rps)])
        pltpu.sync_copy(ones_h, ones_v)
        plsc.subcore_barrier()

        def load_idx(i, m):
            pltpu.async_copy(dst.at[pl.ds(e0 + i * KCH, KCH)], didx[m],
                             si[m])

        load_idx(0, 0)
        load_idx(1, 1)

        def phase(j, m):
            mp = (m + 2) % 4
            @pl.when(j >= 2)
            def _():
                pltpu.make_async_copy(ones_v, cacc.at[didx[mp]],
                                      ss[mp]).wait()
            @pl.when(j + 2 < nch)
            def _():
                load_idx(j + 2, mp)
            pltpu.make_async_copy(dst.at[pl.ds(0, KCH)], didx[m],
                                  si[m]).wait()
            pltpu.async_copy(ones_v, cacc.at[didx[m]], ss[m], add=True)

        def group(g, carry):
            for m in range(4):
                phase(4 * g + m, m)
            return carry

        lax.fori_loop(0, nch // 4, group, 0)
        for m in ((nch - 2) % 4, (nch - 1) % 4):
            pltpu.make_async_copy(ones_v, cacc.at[didx[m]], ss[m]).wait()
        plsc.subcore_barrier()
        o0 = c * npad + r0
        pltpu.sync_copy(cacc.at[pl.ds(r0, rps)], out.at[pl.ds(o0, rps)])

    return pl.kernel(
        body, mesh=mesh,
        out_type=[jax.ShapeDtypeStruct((2 * npad, HALF), F32)],
        scratch_types=(
            [pltpu.VMEM((KCH,), jnp.int32) for _ in range(4)]
            + [pltpu.VMEM((KCH, HALF), F32)]
            + [pltpu.VMEM_SHARED((npad, HALF), F32)]
            + [pltpu.SemaphoreType.DMA] * 8
        ))


def _t2_body(asb_lo, asb_hi, abs_lo, abs_hi, ass_lo, ass_hi,
             csb, cbs, css, rb, rs, b_sb, b_bs, b_ss, Wl1, Wr1,
             y1o, r1b):
    dsb = jnp.maximum(csb[...][:, 0:1], 1.0)
    dbs = jnp.maximum(cbs[...][:, 0:1], 1.0)
    dss = jnp.maximum(css[...][:, 0:1], 1.0)
    m_sb = jnp.concatenate([asb_lo[...], asb_hi[...]], axis=1) / dsb
    m_bs = jnp.concatenate([abs_lo[...], abs_hi[...]], axis=1) / dbs
    m_ss = jnp.concatenate([ass_lo[...], ass_hi[...]], axis=1) / dss
    h_b1 = jnp.maximum(m_sb + b_sb[...] + rb[...], 0.0)
    h_s1 = jnp.maximum(m_bs + m_ss + b_bs[...] + b_ss[...] + rs[...], 0.0)
    y1o[...] = jnp.dot(h_s1, Wl1[...], preferred_element_type=F32)
    r1b[...] = jnp.dot(h_b1, Wr1[...], preferred_element_type=F32)


def _t2(asb_lo, asb_hi, abs_lo, abs_hi, ass_lo, ass_hi, csb, cbs, css,
        rb, rs, b_sb, b_bs, b_ss, Wl1, Wr1):
    n = rb.shape[0]
    h = rb.shape[1]
    grid = n // ROWB
    half = pl.BlockSpec((ROWB, HALF), lambda i: (i, 0))
    cnt = pl.BlockSpec((ROWB, 16), lambda i: (i, 0))
    return pl.pallas_call(
        _t2_body,
        grid=(grid,),
        in_specs=[half] * 6 + [cnt] * 3 + [_rows(h), _rows(h)]
        + [_full((1, h))] * 3 + [_full((h, h))] * 2,
        out_specs=[_rows(h), _rows(h)],
        out_shape=[jax.ShapeDtypeStruct((n, h), F32)] * 2,
    )(asb_lo, asb_hi, abs_lo, abs_hi, ass_lo, ass_hi, csb, cbs, css,
      rb, rs, b_sb, b_bs, b_ss, Wl1, Wr1)


def _t3_body(a1lo, a1hi, csb, r1b, b_sb1, W1, b1, w2row, b2, out):
    d = jnp.maximum(csb[...][:, 0:1], 1.0)
    nb1 = jnp.concatenate([a1lo[...], a1hi[...]], axis=1) / d \
        + b_sb1[...] + r1b[...]
    hidden = jnp.maximum(
        jnp.dot(nb1, W1[...], preferred_element_type=F32) + b1[...], 0.0)
    out[...] = jnp.sum(hidden * w2row[...], axis=1, keepdims=True) + b2[...]


def _t3(a1lo, a1hi, csb, r1b, b_sb1, W1, b1, w2row, b2):
    n = r1b.shape[0]
    h = r1b.shape[1]
    hh = W1.shape[1]
    grid = n // ROWB
    half = pl.BlockSpec((ROWB, HALF), lambda i: (i, 0))
    cnt = pl.BlockSpec((ROWB, 16), lambda i: (i, 0))
    return pl.pallas_call(
        _t3_body,
        grid=(grid,),
        in_specs=[half, half, cnt, _rows(h), _full((1, h)),
                  _full((h, hh)), _full((1, hh)), _full((1, hh)),
                  _full((1, 1))],
        out_specs=pl.BlockSpec((ROWB, 1), lambda i: (i, 0)),
        out_shape=jax.ShapeDtypeStruct((n, 1), F32),
    )(a1lo, a1hi, csb, r1b, b_sb1, W1, b1, w2row, b2)


def kernel(x_bridge, x_street, edge_index_street_to_bridge,
           edge_index_bridge_to_street, edge_index_street_to_street,
           We_b, be_b, We_s, be_s,
           Wl0_sb, bl0_sb, Wr0_sb, Wl0_bs, bl0_bs, Wr0_bs,
           Wl0_ss, bl0_ss, Wr0_ss,
           Wl1_sb, bl1_sb, Wr1_sb, Wl1_bs, bl1_bs, Wr1_bs,
           Wl1_ss, bl1_ss, Wr1_ss,
           W1, b1, W2, b2):
    n_b = x_bridge.shape[0]
    n_s = x_street.shape[0]
    n_e = edge_index_street_to_bridge.shape[1]
    h = We_b.shape[1]

    ep = -(-n_e // (2 * NSUB * KCH)) * (2 * NSUB * KCH)
    pad = ep - n_e
    ei32 = lambda e: e.astype(jnp.int32)
    def sx(e):
        s = jnp.concatenate([ei32(e[0]), jnp.zeros((pad,), jnp.int32)])
        return jnp.concatenate([s * 2, s * 2 + 1])

    def dx(e, trash):
        return jnp.concatenate(
            [ei32(e[1]), jnp.full((pad,), trash, jnp.int32)])

    tr_b = n_b
    tr_s = n_s
    src_sb = sx(edge_index_street_to_bridge)
    dst_sb = dx(edge_index_street_to_bridge, tr_b)
    src_bs = sx(edge_index_bridge_to_street)
    dst_bs = dx(edge_index_bridge_to_street, tr_s)
    src_ss = sx(edge_index_street_to_street)
    dst_ss = dx(edge_index_street_to_street, tr_s)

    row2 = lambda v: v.reshape(1, -1)
    Wr0_s = Wr0_bs + Wr0_ss

    ysb, ybs, yss, rb, rs = _t1(
        x_bridge, x_street, We_b, row2(be_b), We_s, row2(be_s),
        Wl0_sb, Wl0_bs, Wl0_ss, Wr0_sb, Wr0_s)

    npad_b = _pad_rows(n_b)
    npad_s = _pad_rows(n_s)
    zrows = jnp.zeros((max(npad_b, npad_s), HALF), F32)
    ones_h = jnp.ones((KCH, HALF), F32)
    tab = lambda y: y.reshape(-1, HALF)

    agg_b = _make_agg(n_b, ep)
    agg_s = _make_agg(n_s, ep)
    cnt_b = _make_cnt(n_b, ep)
    cnt_s = _make_cnt(n_s, ep)

    csb_p = cnt_b(dst_sb, zrows[:npad_b], ones_h)[0]
    cbs_p = cnt_s(dst_bs, zrows[:npad_s], ones_h)[0]
    css_p = cnt_s(dst_ss, zrows[:npad_s], ones_h)[0]
    asb = agg_b(tab(ysb), src_sb, dst_sb, zrows[:npad_b])[0]
    abs_ = agg_s(tab(ybs), src_bs, dst_bs, zrows[:npad_s])[0]
    ass = agg_s(tab(yss), src_ss, dst_ss, zrows[:npad_s])[0]

    comb = lambda p, npad, n: (p[:n, :16] + p[npad:npad + n, :16])
    csb = comb(csb_p, npad_b, n_b)
    cbs = comb(cbs_p, npad_s, n_s)
    css = comb(css_p, npad_s, n_s)

    lo = lambda a, npad, n: a[:n]
    hi = lambda a, npad, n: a[npad:npad + n]

    y1, r1b = _t2(lo(asb, npad_b, n_b), hi(asb, npad_b, n_b),
                  lo(abs_, npad_s, n_s), hi(abs_, npad_s, n_s),
                  lo(ass, npad_s, n_s), hi(ass, npad_s, n_s),
                  csb, cbs, css, rb, rs,
                  row2(bl0_sb), row2(bl0_bs), row2(bl0_ss),
                  Wl1_sb, Wr1_sb)

    a1 = agg_b(tab(y1), src_sb, dst_sb, zrows[:npad_b])[0]

    return _t3(lo(a1, npad_b, n_b), hi(a1, npad_b, n_b), csb, r1b,
               row2(bl1_sb), W1, row2(b1), W2.reshape(1, -1),
               b2.reshape(1, 1))


# speedup vs baseline: 1.0390x; 1.0093x over previous
import functools

import jax
import jax.numpy as jnp
from jax import lax
from jax.experimental import pallas as pl
from jax.experimental.pallas import tpu as pltpu
from jax.experimental.pallas import tpu_sc as plsc

F32 = jnp.float32
NSUB = 16
KCH = 64
HALF = 128
ROWB = 400


def _full(shape):
    return pl.BlockSpec(shape, lambda i: (0, 0))


def _rows(width):
    return pl.BlockSpec((ROWB, width), lambda i: (i, 0))


def _t1_body(xb, xs, Web, beb, Wes, bes, Wsb, Wbs, Wss, Wrb, Wrs,
             ysb_lo, ybs_lo, yss_lo, rb, rs):
    h_b = jnp.maximum(
        jnp.dot(xb[...], Web[...], preferred_element_type=F32) + beb[...], 0.0)
    h_s = jnp.maximum(
        jnp.dot(xs[...], Wes[...], preferred_element_type=F32) + bes[...], 0.0)
    ysb_lo[...] = jnp.dot(h_s, Wsb[...], preferred_element_type=F32)
    ybs_lo[...] = jnp.dot(h_b, Wbs[...], preferred_element_type=F32)
    yss_lo[...] = jnp.dot(h_s, Wss[...], preferred_element_type=F32)
    rb[...] = jnp.dot(h_b, Wrb[...], preferred_element_type=F32)
    rs[...] = jnp.dot(h_s, Wrs[...], preferred_element_type=F32)


def _t1(xb, xs, Web, beb, Wes, bes, Wsb, Wbs, Wss, Wrb, Wrs):
    n, db = xb.shape
    ds_ = xs.shape[1]
    h = Web.shape[1]
    grid = n // ROWB
    return pl.pallas_call(
        _t1_body,
        grid=(grid,),
        in_specs=[
            _rows(db), _rows(ds_),
            _full((db, h)), _full((1, h)), _full((ds_, h)), _full((1, h)),
            _full((h, h)), _full((h, h)), _full((h, h)),
            _full((h, h)), _full((h, h)),
        ],
        out_specs=[_rows(h)] * 5,
        out_shape=[jax.ShapeDtypeStruct((n, h), F32)] * 5,
    )(xb, xs, Web, beb, Wes, bes, Wsb, Wbs, Wss, Wrb, Wrs)


def _pad_rows(n):
    return ((n + NSUB * 8 - 1) // (NSUB * 8)) * (NSUB * 8)


def _make_agg(n_dst, n_edges):
    e_per_sub = n_edges // NSUB
    nch = e_per_sub // KCH
    assert e_per_sub * NSUB == n_edges and nch * KCH == e_per_sub
    assert nch % 4 == 0
    npad = _pad_rows(n_dst)
    assert npad > n_dst
    rps = npad // NSUB
    mesh = plsc.VectorSubcoreMesh(core_axis_name="c", subcore_axis_name="s")

    assert nch % 8 == 0

    def body(y2, src, dst, zrows, out, *refs):
        idx = refs[0:8]
        didx = refs[8:16]
        rows = refs[16:20]
        accum = refs[20]
        si = refs[21:29]
        sg = refs[29:33]
        ss = refs[33:35]
        c = lax.axis_index("c")
        s = lax.axis_index("s")
        r0 = s * rps
        e0 = s * e_per_sub
        sbase = c * n_edges + e0
        pltpu.sync_copy(zrows.at[pl.ds(r0, rps)], accum.at[pl.ds(r0, rps)])
        plsc.subcore_barrier()

        def load_idx(i, m):
            pltpu.async_copy(src.at[pl.ds(sbase + i * KCH, KCH)], idx[m],
                             si[m])
            pltpu.async_copy(dst.at[pl.ds(e0 + i * KCH, KCH)], didx[m],
                             si[m])

        def wait_idx(m):
            pltpu.make_async_copy(src.at[pl.ds(0, KCH)], idx[m],
                                  si[m]).wait()
            pltpu.make_async_copy(dst.at[pl.ds(0, KCH)], didx[m],
                                  si[m]).wait()

        def gather(m):
            pltpu.async_copy(y2.at[idx[m % 8]], rows[m % 4], sg[m % 4])

        def wait_gather(m):
            pltpu.make_async_copy(y2.at[idx[m % 8]], rows[m % 4],
                                  sg[m % 4]).wait()

        def drain_scatter(m):
            pltpu.make_async_copy(rows[m % 4], accum.at[didx[m % 8]],
                                  ss[m % 2]).wait()

        for m in range(4):
            load_idx(m, m)
        wait_idx(0)
        gather(0)
        wait_idx(1)
        gather(1)

        def phase(j, m):
            @pl.when(j >= 2)
            def _():
                drain_scatter(m + 6)
            @pl.when(j + 2 < nch)
            def _():
                wait_idx((m + 2) % 8)
                gather(m + 2)
            wait_gather(m)
            pltpu.async_copy(rows[m % 4], accum.at[didx[m]], ss[m % 2],
                             add=True)
            @pl.when(j + 4 < nch)
            def _():
                load_idx(j + 4, (m + 4) % 8)

        def group(g, carry):
            for m in range(8):
                phase(8 * g + m, m)
            return carry

        lax.fori_loop(0, nch // 8, group, 0)
        drain_scatter((nch - 2) % 8)
        drain_scatter((nch - 1) % 8)
        plsc.subcore_barrier()
        o0 = c * npad + r0
        pltpu.sync_copy(accum.at[pl.ds(r0, rps)], out.at[pl.ds(o0, rps)])

    return pl.kernel(
        body, mesh=mesh,
        out_type=[jax.ShapeDtypeStruct((2 * npad, HALF), F32)],
        scratch_types=(
            [pltpu.VMEM((KCH,), jnp.int32) for _ in range(16)]
            + [pltpu.VMEM((KCH, HALF), F32) for _ in range(4)]
            + [pltpu.VMEM_SHARED((npad, HALF), F32)]
            + [pltpu.SemaphoreType.DMA] * 8
            + [pltpu.SemaphoreType.DMA] * 4
            + [pltpu.SemaphoreType.DMA] * 2
        ))


def _make_cnt(n_dst, n_edges):
    e_half = n_edges // 2
    e_per_sub = e_half // NSUB
    nch = e_per_sub // KCH
    assert e_per_sub * NSUB * 2 == n_edges and nch * KCH == e_per_sub
    assert nch % 4 == 0
    npad = _pad_rows(n_dst)
    assert npad > n_dst
    rps = npad // NSUB
    mesh = plsc.VectorSubcoreMesh(core_axis_name="c", subcore_axis_name="s")

    def body(dst, zrows, ones_h, out, *refs):
        didx = refs[0:4]
        ones_v = refs[4]
        cacc = refs[5]
        si = refs[6:10]
        ss = refs[10:14]
        c = lax.axis_index("c")
        s = lax.axis_index("s")
        r0 = s * rps
        e0 = c * e_half + s * e_per_sub
        pltpu.sync_copy(zrows.at[pl.ds(r0, rps)], cacc.at[pl.ds(r0, rps)])
        pltpu.sync_copy(ones_h, ones_v)
        plsc.subcore_barrier()

        def load_idx(i, m):
            pltpu.async_copy(dst.at[pl.ds(e0 + i * KCH, KCH)], didx[m],
                             si[m])

        load_idx(0, 0)
        load_idx(1, 1)

        def phase(j, m):
            mp = (m + 2) % 4
            @pl.when(j >= 2)
            def _():
                pltpu.make_async_copy(ones_v, cacc.at[didx[mp]],
                                      ss[mp]).wait()
            @pl.when(j + 2 < nch)
            def _():
                load_idx(j + 2, mp)
            pltpu.make_async_copy(dst.at[pl.ds(0, KCH)], didx[m],
                                  si[m]).wait()
            pltpu.async_copy(ones_v, cacc.at[didx[m]], ss[m], add=True)

        def group(g, carry):
            for m in range(4):
                phase(4 * g + m, m)
            return carry

        lax.fori_loop(0, nch // 4, group, 0)
        for m in ((nch - 2) % 4, (nch - 1) % 4):
            pltpu.make_async_copy(ones_v, cacc.at[didx[m]], ss[m]).wait()
        plsc.subcore_barrier()
        o0 = c * npad + r0
        pltpu.sync_copy(cacc.at[pl.ds(r0, rps)], out.at[pl.ds(o0, rps)])

    return pl.kernel(
        body, mesh=mesh,
        out_type=[jax.ShapeDtypeStruct((2 * npad, HALF), F32)],
        scratch_types=(
            [pltpu.VMEM((KCH,), jnp.int32) for _ in range(4)]
            + [pltpu.VMEM((KCH, HALF), F32)]
            + [pltpu.VMEM_SHARED((npad, HALF), F32)]
            + [pltpu.SemaphoreType.DMA] * 8
        ))


def _t2_body(asb_lo, asb_hi, abs_lo, abs_hi, ass_lo, ass_hi,
             csb, cbs, css, rb, rs, b_sb, b_bs, b_ss, Wl1, Wr1,
             y1o, r1b):
    dsb = jnp.maximum(csb[...][:, 0:1], 1.0)
    dbs = jnp.maximum(cbs[...][:, 0:1], 1.0)
    dss = jnp.maximum(css[...][:, 0:1], 1.0)
    m_sb = jnp.concatenate([asb_lo[...], asb_hi[...]], axis=1) / dsb
    m_bs = jnp.concatenate([abs_lo[...], abs_hi[...]], axis=1) / dbs
    m_ss = jnp.concatenate([ass_lo[...], ass_hi[...]], axis=1) / dss
    h_b1 = jnp.maximum(m_sb + b_sb[...] + rb[...], 0.0)
    h_s1 = jnp.maximum(m_bs + m_ss + b_bs[...] + b_ss[...] + rs[...], 0.0)
    y1o[...] = jnp.dot(h_s1, Wl1[...], preferred_element_type=F32)
    r1b[...] = jnp.dot(h_b1, Wr1[...], preferred_element_type=F32)


def _t2(asb_lo, asb_hi, abs_lo, abs_hi, ass_lo, ass_hi, csb, cbs, css,
        rb, rs, b_sb, b_bs, b_ss, Wl1, Wr1):
    n = rb.shape[0]
    h = rb.shape[1]
    grid = n // ROWB
    half = pl.BlockSpec((ROWB, HALF), lambda i: (i, 0))
    cnt = pl.BlockSpec((ROWB, 16), lambda i: (i, 0))
    return pl.pallas_call(
        _t2_body,
        grid=(grid,),
        in_specs=[half] * 6 + [cnt] * 3 + [_rows(h), _rows(h)]
        + [_full((1, h))] * 3 + [_full((h, h))] * 2,
        out_specs=[_rows(h), _rows(h)],
        out_shape=[jax.ShapeDtypeStruct((n, h), F32)] * 2,
    )(asb_lo, asb_hi, abs_lo, abs_hi, ass_lo, ass_hi, csb, cbs, css,
      rb, rs, b_sb, b_bs, b_ss, Wl1, Wr1)


def _t3_body(a1lo, a1hi, csb, r1b, b_sb1, W1, b1, w2row, b2, out):
    d = jnp.maximum(csb[...][:, 0:1], 1.0)
    nb1 = jnp.concatenate([a1lo[...], a1hi[...]], axis=1) / d \
        + b_sb1[...] + r1b[...]
    hidden = jnp.maximum(
        jnp.dot(nb1, W1[...], preferred_element_type=F32) + b1[...], 0.0)
    out[...] = jnp.sum(hidden * w2row[...], axis=1, keepdims=True) + b2[...]


def _t3(a1lo, a1hi, csb, r1b, b_sb1, W1, b1, w2row, b2):
    n = r1b.shape[0]
    h = r1b.shape[1]
    hh = W1.shape[1]
    grid = n // ROWB
    half = pl.BlockSpec((ROWB, HALF), lambda i: (i, 0))
    cnt = pl.BlockSpec((ROWB, 16), lambda i: (i, 0))
    return pl.pallas_call(
        _t3_body,
        grid=(grid,),
        in_specs=[half, half, cnt, _rows(h), _full((1, h)),
                  _full((h, hh)), _full((1, hh)), _full((1, hh)),
                  _full((1, 1))],
        out_specs=pl.BlockSpec((ROWB, 1), lambda i: (i, 0)),
        out_shape=jax.ShapeDtypeStruct((n, 1), F32),
    )(a1lo, a1hi, csb, r1b, b_sb1, W1, b1, w2row, b2)


def kernel(x_bridge, x_street, edge_index_street_to_bridge,
           edge_index_bridge_to_street, edge_index_street_to_street,
           We_b, be_b, We_s, be_s,
           Wl0_sb, bl0_sb, Wr0_sb, Wl0_bs, bl0_bs, Wr0_bs,
           Wl0_ss, bl0_ss, Wr0_ss,
           Wl1_sb, bl1_sb, Wr1_sb, Wl1_bs, bl1_bs, Wr1_bs,
           Wl1_ss, bl1_ss, Wr1_ss,
           W1, b1, W2, b2):
    n_b = x_bridge.shape[0]
    n_s = x_street.shape[0]
    n_e = edge_index_street_to_bridge.shape[1]
    h = We_b.shape[1]

    ep = -(-n_e // (8 * NSUB * KCH)) * (8 * NSUB * KCH)
    pad = ep - n_e
    ei32 = lambda e: e.astype(jnp.int32)
    def sx(e):
        s = jnp.concatenate([ei32(e[0]), jnp.zeros((pad,), jnp.int32)])
        return jnp.concatenate([s * 2, s * 2 + 1])

    def dx(e, trash):
        return jnp.concatenate(
            [ei32(e[1]), jnp.full((pad,), trash, jnp.int32)])

    tr_b = n_b
    tr_s = n_s
    src_sb = sx(edge_index_street_to_bridge)
    dst_sb = dx(edge_index_street_to_bridge, tr_b)
    src_bs = sx(edge_index_bridge_to_street)
    dst_bs = dx(edge_index_bridge_to_street, tr_s)
    src_ss = sx(edge_index_street_to_street)
    dst_ss = dx(edge_index_street_to_street, tr_s)

    row2 = lambda v: v.reshape(1, -1)
    Wr0_s = Wr0_bs + Wr0_ss

    ysb, ybs, yss, rb, rs = _t1(
        x_bridge, x_street, We_b, row2(be_b), We_s, row2(be_s),
        Wl0_sb, Wl0_bs, Wl0_ss, Wr0_sb, Wr0_s)

    npad_b = _pad_rows(n_b)
    npad_s = _pad_rows(n_s)
    zrows = jnp.zeros((max(npad_b, npad_s), HALF), F32)
    ones_h = jnp.ones((KCH, HALF), F32)
    tab = lambda y: y.reshape(-1, HALF)

    agg_b = _make_agg(n_b, ep)
    agg_s = _make_agg(n_s, ep)
    cnt_b = _make_cnt(n_b, ep)
    cnt_s = _make_cnt(n_s, ep)

    csb_p = cnt_b(dst_sb, zrows[:npad_b], ones_h)[0]
    cbs_p = cnt_s(dst_bs, zrows[:npad_s], ones_h)[0]
    css_p = cnt_s(dst_ss, zrows[:npad_s], ones_h)[0]
    asb = agg_b(tab(ysb), src_sb, dst_sb, zrows[:npad_b])[0]
    abs_ = agg_s(tab(ybs), src_bs, dst_bs, zrows[:npad_s])[0]
    ass = agg_s(tab(yss), src_ss, dst_ss, zrows[:npad_s])[0]

    comb = lambda p, npad, n: (p[:n, :16] + p[npad:npad + n, :16])
    csb = comb(csb_p, npad_b, n_b)
    cbs = comb(cbs_p, npad_s, n_s)
    css = comb(css_p, npad_s, n_s)

    lo = lambda a, npad, n: a[:n]
    hi = lambda a, npad, n: a[npad:npad + n]

    y1, r1b = _t2(lo(asb, npad_b, n_b), hi(asb, npad_b, n_b),
                  lo(abs_, npad_s, n_s), hi(abs_, npad_s, n_s),
                  lo(ass, npad_s, n_s), hi(ass, npad_s, n_s),
                  csb, cbs, css, rb, rs,
                  row2(bl0_sb), row2(bl0_bs), row2(bl0_ss),
                  Wl1_sb, Wr1_sb)

    a1 = agg_b(tab(y1), src_sb, dst_sb, zrows[:npad_b])[0]

    return _t3(lo(a1, npad_b, n_b), hi(a1, npad_b, n_b), csb, r1b,
               row2(bl1_sb), W1, row2(b1), W2.reshape(1, -1),
               b2.reshape(1, 1))
